# baseline (speedup 1.0000x reference)
import jax
import jax.numpy as jnp
from jax import lax
from jax.experimental import pallas as pl

B, F, N, K = 8, 64, 256, 512


def _split(a):
    hi = a.astype(jnp.bfloat16)
    lo = (a - hi.astype(jnp.float32)).astype(jnp.bfloat16)
    return hi, lo


def _bdot(a, b, dims):
    return lax.dot_general(a, b, (dims, ((), ())),
                           preferred_element_type=jnp.float32)


G = 4


def _vq_body(x_ref, emb_ref, q_ref, dist_ref):
  emb = emb_ref[...]
  eh, el = _split(emb)
  c2 = jnp.sum(emb * emb, axis=1, keepdims=True)
  for bi in range(G):
    xb = x_ref[bi]
    xh, xl = _split(xb)
    cd = ((1,), (0,))
    dotT = ((_bdot(eh, xh, cd) + _bdot(el, xl, cd))
            + (_bdot(eh, xl, cd) + _bdot(el, xh, cd)))
    x2 = jnp.sum(xb * xb, axis=0, keepdims=True)
    gT = c2 - 2.0 * dotT
    dist_ref[bi] = (gT + x2).T
    minv = jnp.min(gT, axis=0, keepdims=True)
    iota = lax.broadcasted_iota(jnp.int32, (K, N), 0)
    idx = jnp.min(jnp.where(gT == minv, iota, K), axis=0, keepdims=True)
    ohT = (iota == idx).astype(jnp.bfloat16)
    cq = ((0,), (0,))
    q_ref[bi] = _bdot(eh, ohT, cq) + _bdot(el, ohT, cq)


def kernel(x, emb_weight):
    q, dist = pl.pallas_call(
        _vq_body,
        grid=(B // G,),
        in_specs=[
            pl.BlockSpec((G, F, N), lambda b: (b, 0, 0)),
            pl.BlockSpec((K, F), lambda b: (0, 0)),
        ],
        out_specs=[
            pl.BlockSpec((G, F, N), lambda b: (b, 0, 0)),
            pl.BlockSpec((G, N, K), lambda b: (b, 0, 0)),
        ],
        out_shape=[
            jax.ShapeDtypeStruct((B, F, N), jnp.float32),
            jax.ShapeDtypeStruct((B, N, K), jnp.float32),
        ],
    )(x, emb_weight)
    return q, dist
